# baseline (device time: 25936 ns/iter reference)
import jax
import jax.numpy as jnp
from jax import lax
from jax.experimental import pallas as pl
from jax.experimental.pallas import tpu as pltpu

N_DEV = 4
M = 1024
N_FULL = 2048
N_PER = N_FULL // N_DEV
HALF = N_PER // 2
ROW_K = 4
SEG_M = M // ROW_K
N_HOPS = N_DEV - 1
DIRS = (+1, -1)


def kernel(x):
    chains = []
    for j in range(ROW_K):
        chains.append((0, 0, j * SEG_M))
        chains.append((1, HALF, j * SEG_M))
    n_chains = len(chains)

    def body(x_ref, out_ref, comm_ref, stage_ref, send_sems, recv_sems,
             stage_sems):
        my = lax.axis_index("i")
        left = lax.rem(my + N_DEV - 1, N_DEV)
        right = lax.rem(my + 1, N_DEV)

        def chunk_for(di, phase):
            d = DIRS[di]
            k = 1 if phase == 0 else phase + 1
            return lax.rem(my + 2 * N_DEV - d * k, N_DEV)

        def stage_dma(di, phase):
            col = chunk_for(di, phase) * N_PER + di * HALF
            return pltpu.make_async_copy(
                x_ref.at[0, :, pl.ds(col, HALF)],
                stage_ref.at[phase, di],
                stage_sems.at[phase, di],
            )

        def rdma(c, s):
            di = chains[c][0]
            return pltpu.make_async_remote_copy(
                src_ref=comm_ref.at[c, s],
                dst_ref=comm_ref.at[c, s + 1],
                send_sem=send_sems.at[c, s],
                recv_sem=recv_sems.at[c, s],
                device_id=(right if di == 0 else left,),
                device_id_type=pl.DeviceIdType.MESH,
            )

        def staged_seg(phase, di, ro):
            return stage_ref[phase, di, pl.ds(ro, SEG_M), :].astype(
                jnp.bfloat16
            )

        barrier_sem = pltpu.get_barrier_semaphore()
        for nbr in (left, right):
            pl.semaphore_signal(
                barrier_sem, inc=1,
                device_id=(nbr,), device_id_type=pl.DeviceIdType.MESH,
            )
        stage_dma(0, 0).start()
        stage_dma(1, 0).start()
        pl.semaphore_wait(barrier_sem, 2)

        for di in range(2):
            stage_dma(di, 0).wait()
            for c, (cdi, co, ro) in enumerate(chains):
                if cdi == di:
                    comm_ref[c, 0, :, :] = staged_seg(0, di, ro)
                    rdma(c, 0).start()

        for s in range(N_HOPS):
            for di in range(2):
                stage_dma(di, s + 1).start()

        for s in range(N_HOPS):
            waited = [False, False]
            for c, (di, co, ro) in enumerate(chains):
                if not waited[di]:
                    stage_dma(di, s + 1).wait()
                    waited[di] = True
                rdma(c, s).wait_recv()
                comm_ref[c, s + 1, :, :] += staged_seg(s + 1, di, ro)
                if s + 1 < N_HOPS:
                    rdma(c, s + 1).start()
                else:
                    out_ref[pl.ds(ro, SEG_M), pl.ds(co, HALF)] = comm_ref[
                        c, N_HOPS, :, :
                    ]

        for s in range(N_HOPS):
            for c in range(n_chains):
                rdma(c, s).wait_send()

    return pl.pallas_call(
        body,
        out_shape=jax.ShapeDtypeStruct((M, N_PER), jnp.bfloat16),
        in_specs=[pl.BlockSpec(memory_space=pl.ANY)],
        out_specs=pl.BlockSpec(memory_space=pltpu.VMEM),
        scratch_shapes=[
            pltpu.VMEM((n_chains, N_DEV, SEG_M, HALF), jnp.bfloat16),
            pltpu.VMEM((N_DEV, 2, M, HALF), jnp.float32),
            pltpu.SemaphoreType.DMA((n_chains, N_HOPS)),
            pltpu.SemaphoreType.DMA((n_chains, N_HOPS)),
            pltpu.SemaphoreType.DMA((N_DEV, 2)),
        ],
        compiler_params=pltpu.CompilerParams(
            collective_id=0,
            vmem_limit_bytes=100 * 1024 * 1024,
        ),
    )(x)


# device time: 23614 ns/iter; 1.0983x vs baseline; 1.0983x over previous
import jax
import jax.numpy as jnp
from jax import lax
from jax.experimental import pallas as pl
from jax.experimental.pallas import tpu as pltpu

N_DEV = 4
M = 1024
N_FULL = 2048
N_PER = N_FULL // N_DEV
HALF = N_PER // 2
ROW_K = 4
SEG_M = M // ROW_K
N_HOPS = N_DEV - 1
DIRS = (+1, -1)


def kernel(x):
    chains = []
    for j in range(ROW_K):
        chains.append((0, 0, j * SEG_M))
        chains.append((1, HALF, j * SEG_M))
    n_chains = len(chains)

    def body(x_ref, out_ref, comm_ref, stage_ref, send_sems, recv_sems,
             stage_sems):
        my = lax.axis_index("i")
        left = lax.rem(my + N_DEV - 1, N_DEV)
        right = lax.rem(my + 1, N_DEV)

        def chunk_for(di, phase):
            d = DIRS[di]
            k = 1 if phase == 0 else phase + 1
            return lax.rem(my + 2 * N_DEV - d * k, N_DEV)

        def stage_dma(di, phase):
            col = chunk_for(di, phase) * N_PER + di * HALF
            return pltpu.make_async_copy(
                x_ref.at[0, :, pl.ds(col, HALF)],
                stage_ref.at[phase, di],
                stage_sems.at[phase, di],
            )

        def rdma(c, s):
            di = chains[c][0]
            return pltpu.make_async_remote_copy(
                src_ref=comm_ref.at[c, s],
                dst_ref=comm_ref.at[c, s + 1],
                send_sem=send_sems.at[c, s],
                recv_sem=recv_sems.at[c, s],
                device_id=(right if di == 0 else left,),
                device_id_type=pl.DeviceIdType.MESH,
            )

        def staged_seg(phase, di, ro):
            return stage_ref[phase, di, pl.ds(ro, SEG_M), :].astype(
                jnp.bfloat16
            )

        barrier_sem = pltpu.get_barrier_semaphore()
        for nbr in (left, right):
            pl.semaphore_signal(
                barrier_sem, inc=1,
                device_id=(nbr,), device_id_type=pl.DeviceIdType.MESH,
            )
        stage_dma(0, 0).start()
        stage_dma(1, 0).start()
        pl.semaphore_wait(barrier_sem, 2)

        for di in range(2):
            stage_dma(di, 0).wait()
            for c, (cdi, co, ro) in enumerate(chains):
                if cdi == di:
                    comm_ref[c, 0, :, :] = staged_seg(0, di, ro)
                    rdma(c, 0).start()

        for s in range(N_HOPS):
            for di in range(2):
                stage_dma(di, s + 1).start()

        for s in range(N_HOPS):
            waited = [False, False]
            for c, (di, co, ro) in enumerate(chains):
                if not waited[di]:
                    stage_dma(di, s + 1).wait()
                    waited[di] = True
                rdma(c, s).wait_recv()
                comm_ref[c, s + 1, :, :] += staged_seg(s + 1, di, ro)
                if s + 1 < N_HOPS:
                    rdma(c, s + 1).start()
                else:
                    out_ref[pl.ds(ro, SEG_M), pl.ds(co, HALF)] = comm_ref[
                        c, N_HOPS, :, :
                    ]

        for s in range(N_HOPS):
            for c in range(n_chains):
                rdma(c, s).wait_send()

    return pl.pallas_call(
        body,
        out_shape=jax.ShapeDtypeStruct((M, N_PER), jnp.bfloat16),
        in_specs=[pl.BlockSpec(memory_space=pl.ANY)],
        out_specs=pl.BlockSpec(memory_space=pltpu.VMEM),
        scratch_shapes=[
            pltpu.VMEM((n_chains, N_DEV, SEG_M, HALF), jnp.bfloat16),
            pltpu.VMEM((N_DEV, 2, M, HALF), jnp.float32),
            pltpu.SemaphoreType.DMA((n_chains, N_HOPS)),
            pltpu.SemaphoreType.DMA((n_chains, N_HOPS)),
            pltpu.SemaphoreType.DMA((N_DEV, 2)),
        ],
        compiler_params=pltpu.CompilerParams(
            collective_id=0,
            vmem_limit_bytes=52 * 1024 * 1024,
        ),
    )(x)
